# baseline (speedup 1.0000x reference)
import functools

import jax
import jax.numpy as jnp
from jax import lax
from jax.experimental import pallas as pl
from jax.experimental.pallas import tpu as pltpu
from jax.experimental.pallas import tpu_sc as plsc

N = 10000
E = 320000
DS = 128
DV = 32

NC = 2
NS = 16
CH = 80

NP = 10240
EP = 4096 * CH
E0 = 204800
E1 = E - E0
EPB = EP - E0


def _dotT(x, w):
    return lax.dot_general(x, w, (((1,), (1,)), ((), ())),
                           preferred_element_type=jnp.float32)


def _dot(x, w):
    return lax.dot_general(x, w, (((1,), (0,)), ((), ())),
                           preferred_element_type=jnp.float32)


def _proj_body(s_ref, wa_ref, wb_ref, sa_ref, sb_ref):
    x = s_ref[...]
    sa_ref[...] = _dotT(x, wa_ref[...])
    sb_ref[...] = _dotT(x, wb_ref[...])


def _node_proj(s, W1a, W1b):
    B = 1000
    return pl.pallas_call(
        _proj_body,
        grid=(N // B,),
        in_specs=[pl.BlockSpec((B, DS), lambda i: (i, 0)),
                  pl.BlockSpec((DS, DS), lambda i: (0, 0)),
                  pl.BlockSpec((DS, DS), lambda i: (0, 0))],
        out_specs=[pl.BlockSpec((B, DS), lambda i: (i, 0)),
                   pl.BlockSpec((B, DS), lambda i: (i, 0))],
        out_shape=[jax.ShapeDtypeStruct((N, DS), jnp.float32),
                   jax.ShapeDtypeStruct((N, DS), jnp.float32)],
    )(s, W1a, W1b)


def _gather(sa, sb, row, col, e0, esz):
    mesh = plsc.VectorSubcoreMesh(core_axis_name="c", subcore_axis_name="s")
    per_w = esz // (NC * NS)
    n_ch = per_w // CH
    n_quad = n_ch // 4
    n_tail = n_ch % 4
    NR = 4

    @functools.partial(
        pl.kernel, mesh=mesh,
        out_type=jax.ShapeDtypeStruct((esz, DS), jnp.float32),
        scratch_types=[
            pltpu.VMEM((per_w,), jnp.int32),
            pltpu.VMEM((per_w,), jnp.int32),
            [pltpu.VMEM((CH, DS), jnp.float32) for _ in range(NR)],
            [pltpu.VMEM((CH, DS), jnp.float32) for _ in range(NR)],
            [pltpu.SemaphoreType.DMA for _ in range(NR)],
            [pltpu.SemaphoreType.DMA for _ in range(NR)],
        ],
    )
    def k(sa_hbm, sb_hbm, row_hbm, col_hbm, g_hbm,
          idxr, idxc, abufs, bbufs, insems, wrsems):
        cid = lax.axis_index("c")
        sid = lax.axis_index("s")
        wid = sid * NC + cid
        base = wid * per_w
        pltpu.sync_copy(row_hbm.at[pl.ds(e0 + base, per_w)], idxr)
        pltpu.sync_copy(col_hbm.at[pl.ds(e0 + base, per_w)], idxc)

        def issue(i, r):
            pltpu.async_copy(sa_hbm.at[idxr.at[pl.ds(i * CH, CH)]],
                             abufs[r], insems[r])
            pltpu.async_copy(sb_hbm.at[idxc.at[pl.ds(i * CH, CH)]],
                             bbufs[r], insems[r])

        def wait_in(r):
            pltpu.make_async_copy(sa_hbm.at[pl.ds(0, CH)], abufs[r],
                                  insems[r]).wait()
            pltpu.make_async_copy(sa_hbm.at[pl.ds(0, CH)], bbufs[r],
                                  insems[r]).wait()

        def wait_wr(r):
            pltpu.make_async_copy(abufs[r], g_hbm.at[pl.ds(0, CH)],
                                  wrsems[r]).wait()

        def add_write(i, r):
            abuf, bbuf = abufs[r], bbufs[r]

            def add_row(rr, c2):
                for c8 in range(DS // 16):
                    sl = pl.ds(c8 * 16, 16)
                    abuf[rr, sl] = abuf[rr, sl] + bbuf[rr, sl]
                return c2
            lax.fori_loop(0, CH, add_row, 0, unroll=4)
            pltpu.async_copy(abuf, g_hbm.at[pl.ds(base + i * CH, CH)],
                             wrsems[r])

        for r in range(NR - 1):
            issue(r, r)

        def body(t, carry):
            c = NR * t
            for kk in range(NR):
                r = kk
                rn = (kk + NR - 1) % NR
                wait_in(r)
                add_write(c + kk, r)
                ck = c + kk

                @pl.when((ck > 0) & (ck + NR - 1 < n_ch))
                def _():
                    wait_wr(rn)

                @pl.when(ck + NR - 1 < n_ch)
                def _():
                    issue(ck + NR - 1, rn)
            return carry
        lax.fori_loop(0, n_quad, body, 0)

        for ck in range(n_quad * NR, n_ch):
            wait_in(ck % NR)
            add_write(ck, ck % NR)
        for r in range(NR):
            wait_wr(r)

    return k(sa, sb, row, col)


def _edge_body_sz(nlive, g_ref, ea_ref, un_ref, w1c_ref, b1_ref, w2_ref,
                  b2_ref, wvb_ref, bvb_ref, p_ref, m_ref, w_ref):
    h = g_ref[...] + _dotT(ea_ref[...], w1c_ref[...]) + b1_ref[...]
    h = h * jax.nn.sigmoid(h)
    m = _dotT(h, w2_ref[...]) + b2_ref[...]
    sm = m * jax.nn.sigmoid(m)
    w = _dot(sm, wvb_ref[...]) + bvb_ref[...]
    u = _dot(un_ref[...], p_ref[...])
    B = m.shape[0]
    erow = pl.program_id(0) * B + lax.broadcasted_iota(jnp.int32, (B, 1), 0)
    live = erow < nlive
    m_ref[...] = jnp.where(live, m, 0.0)
    w_ref[...] = jnp.where(live, w * u, 0.0)


def _edge_mlp(g, ea, unit, W1c, b1, W2, b2, Wv_int, bv_int, P, blk0, esz, epad):
    B = 1024
    nlive_blk = (esz - 1) // B
    clampg = lambda i: (jnp.minimum(i, nlive_blk), 0)
    clampo = lambda i: (jnp.minimum(i, nlive_blk) + blk0, 0)
    const = lambda i: (0, 0)
    body = functools.partial(_edge_body_sz, esz)
    return pl.pallas_call(
        body,
        grid=(epad // B,),
        in_specs=[pl.BlockSpec((B, DS), clampg),
                  pl.BlockSpec((B, DS), clampo),
                  pl.BlockSpec((B, 3), clampo),
                  pl.BlockSpec((DS, DS), const),
                  pl.BlockSpec((1, DS), const),
                  pl.BlockSpec((DS, DS), const),
                  pl.BlockSpec((1, DS), const),
                  pl.BlockSpec((DS, DS), const),
                  pl.BlockSpec((1, DS), const),
                  pl.BlockSpec((3, DS), const)],
        out_specs=[pl.BlockSpec((B, DS), lambda i: (i, 0)),
                   pl.BlockSpec((B, DS), lambda i: (i, 0))],
        out_shape=[jax.ShapeDtypeStruct((epad, DS), jnp.float32),
                   jax.ShapeDtypeStruct((epad, DS), jnp.float32)],
    )(g, ea, unit, W1c, b1, W2, b2, Wv_int, bv_int, P)


def _scatter_half(m, w, rowp, v2p, c0, n_chunks, prev=None):
    mesh = plsc.VectorSubcoreMesh(core_axis_name="c", subcore_axis_name="s")
    per_t = n_chunks // NS
    n_tri = per_t // 3
    nb = NP // NS
    ZR = 16
    NR = 3
    final = prev is not None

    @functools.partial(
        pl.kernel, mesh=mesh,
        out_type=[jax.ShapeDtypeStruct((NP, DS), jnp.float32),
                  jax.ShapeDtypeStruct((NP, 96), jnp.float32)],
        scratch_types=[
            pltpu.VMEM_SHARED((NP, DS), jnp.float32),
            pltpu.VMEM((NR, CH), jnp.int32),
            [pltpu.VMEM((CH, DS), jnp.float32) for _ in range(NR)],
            pltpu.VMEM((ZR, DS), jnp.float32),
            pltpu.VMEM((ZR, DS), jnp.float32),
            pltpu.VMEM((ZR, 96), jnp.float32),
            pltpu.VMEM((ZR, 96), jnp.float32),
            [pltpu.SemaphoreType.DMA for _ in range(NR)],
            [pltpu.SemaphoreType.DMA for _ in range(NR)],
        ],
    )
    def k(m_hbm, w_hbm, row_hbm, v_hbm, *rest):
        if final:
            sprev_hbm, vprev_hbm = rest[0], rest[1]
            rest = rest[2:]
        (sout_hbm, vnew_hbm, acc, idxs, dats, stage, stage2,
         vstage, vstage2, insems, sssems) = rest
        cid = lax.axis_index("c")
        sid = lax.axis_index("s")
        zero16 = jnp.zeros((16,), jnp.float32)

        def zrow(r, c):
            for c8 in range(DS // 16):
                stage[r, pl.ds(c8 * 16, 16)] = zero16
            return c
        lax.fori_loop(0, ZR, zrow, 0)
        nbase = sid * nb

        def zblk(j, c):
            b0 = pl.multiple_of(nbase + j * ZR, 8)
            pltpu.sync_copy(stage, acc.at[pl.ds(b0, ZR)])
            return c
        lax.fori_loop(0, nb // ZR, zblk, 0)
        plsc.subcore_barrier()

        cbase = sid * per_t

        def issue(lc, r):
            gr = cbase + lc
            pltpu.async_copy(row_hbm.at[pl.ds((c0 + gr) * CH, CH)],
                             idxs.at[r], insems[r])

            @pl.when(cid == 0)
            def _():
                pltpu.async_copy(m_hbm.at[pl.ds(gr * CH, CH)], dats[r],
                                 insems[r])

            @pl.when(cid == 1)
            def _():
                pltpu.async_copy(w_hbm.at[pl.ds(gr * CH, CH)], dats[r],
                                 insems[r])

        def wait_in(r):
            pltpu.make_async_copy(row_hbm.at[pl.ds(0, CH)], idxs.at[r],
                                  insems[r]).wait()
            pltpu.make_async_copy(m_hbm.at[pl.ds(0, CH)], dats[r],
                                  insems[r]).wait()

        def scat(r):
            pltpu.async_copy(dats[r], acc.at[idxs.at[r]], sssems[r], add=True)

        def wait_ss(r):
            pltpu.make_async_copy(dats[r], acc.at[idxs.at[r]],
                                  sssems[r]).wait()

        for r in range(NR - 1):
            issue(r, r)

        def body(t, carry):
            c = NR * t
            for kk in range(NR):
                r = kk
                rn = (kk + NR - 1) % NR
                wait_in(r)
                scat(r)
                ck = c + kk

                @pl.when((ck > 0) & (ck + NR - 1 < per_t))
                def _():
                    wait_ss(rn)

                @pl.when(ck + NR - 1 < per_t)
                def _():
                    issue(ck + NR - 1, rn)
            return carry
        lax.fori_loop(0, n_tri, body, 0)

        for ck in range(n_tri * NR, per_t):
            wait_in(ck % NR)
            scat(ck % NR)
        for r in range(NR):
            wait_ss(r)
        plsc.subcore_barrier()

        @pl.when(cid == 0)
        def _():
            def sblk(j, c):
                b0 = pl.multiple_of(nbase + j * ZR, 8)
                pltpu.sync_copy(acc.at[pl.ds(b0, ZR)], stage)
                if final:
                    pltpu.sync_copy(sprev_hbm.at[pl.ds(b0, ZR)], stage2)

                    def srow(r, c2):
                        for vv in range(DS // 16):
                            sl = pl.ds(vv * 16, 16)
                            stage[r, sl] = stage[r, sl] + stage2[r, sl]
                        return c2
                    lax.fori_loop(0, ZR, srow, 0, unroll=4)
                pltpu.sync_copy(stage, sout_hbm.at[pl.ds(b0, ZR)])
                return c
            lax.fori_loop(0, nb // ZR, sblk, 0)

        @pl.when(cid == 1)
        def _():
            def vblk(j, c):
                b0 = pl.multiple_of(nbase + j * ZR, 8)
                pltpu.sync_copy(acc.at[pl.ds(b0, ZR)], stage)
                pltpu.sync_copy(v_hbm.at[pl.ds(b0, ZR)], vstage)
                if final:
                    pltpu.sync_copy(vprev_hbm.at[pl.ds(b0, ZR)], vstage2)

                def prow(r, c2):
                    for vv in range(6):
                        sl = pl.ds(vv * 16, 16)
                        x = vstage[r, sl] + stage[r, sl]
                        if final:
                            x = x + vstage2[r, sl]
                        vstage[r, sl] = x
                    return c2
                lax.fori_loop(0, ZR, prow, 0, unroll=4)
                pltpu.sync_copy(vstage, vnew_hbm.at[pl.ds(b0, ZR)])
                return c
            lax.fori_loop(0, nb // ZR, vblk, 0)

    if final:
        return k(m, w, rowp, v2p, prev[0], prev[1])
    return k(m, w, rowp, v2p)


def _node_body(s_ref, so_ref, ws_ref, bs_ref, g_ref, b_ref, out_ref):
    so = so_ref[...]
    sm = so * jax.nn.sigmoid(so)
    u = s_ref[...] + _dotT(sm, ws_ref[...]) + bs_ref[...]
    mu = jnp.mean(u, axis=1, keepdims=True)
    d = u - mu
    var = jnp.mean(d * d, axis=1, keepdims=True)
    out_ref[...] = d * lax.rsqrt(var + 1e-5) * g_ref[...] + b_ref[...]


def _node_out(s, s_out, W_s, bs, g, b):
    B = 1000
    return pl.pallas_call(
        _node_body,
        grid=(N // B,),
        in_specs=[pl.BlockSpec((B, DS), lambda i: (i, 0)),
                  pl.BlockSpec((B, DS), lambda i: (i, 0)),
                  pl.BlockSpec((DS, DS), lambda i: (0, 0)),
                  pl.BlockSpec((1, DS), lambda i: (0, 0)),
                  pl.BlockSpec((1, DS), lambda i: (0, 0)),
                  pl.BlockSpec((1, DS), lambda i: (0, 0))],
        out_specs=pl.BlockSpec((B, DS), lambda i: (i, 0)),
        out_shape=jax.ShapeDtypeStruct((N, DS), jnp.float32),
    )(s, s_out, W_s, bs, g, b)


def kernel(s, v, edge_index, edge_attr, edge_vec_unit,
           W_e1, b_e1, W_e2, b_e2, W_s, b_s, W_v, b_v, ln_g, ln_b):
    row = edge_index[0]
    col = edge_index[1]
    W1a = W_e1[:, :DS]
    W1b = W_e1[:, DS:2 * DS]
    W1c = W_e1[:, 2 * DS:]
    WvT = W_v.T
    jidx = jnp.arange(96) // 3
    Wv_int = jnp.concatenate(
        [WvT[:, jidx], jnp.zeros((DS, DV), jnp.float32)], axis=1)
    bv_int = jnp.pad(b_v[jidx], (0, DV)).reshape(1, DS)
    kcol = jnp.arange(96) % 3
    P = jnp.pad((kcol[None, :] == jnp.arange(3)[:, None]).astype(jnp.float32),
                ((0, 0), (0, DV)))

    sa, sb = _node_proj(s, W1a, W1b)
    mlp_w = (W1c, b_e1.reshape(1, DS), W_e2, b_e2.reshape(1, DS),
             Wv_int, bv_int, P)
    g0 = _gather(sa, sb, row, col, 0, E0)
    g1 = _gather(sa, sb, row, col, E0, E1)
    m0, w0 = _edge_mlp(g0, edge_attr, edge_vec_unit, *mlp_w, 0, E0, E0)
    m1, w1 = _edge_mlp(g1, edge_attr, edge_vec_unit, *mlp_w,
                       E0 // 1024, E1, EPB)
    rowp = jnp.concatenate(
        [row[:E0], jnp.pad(row[E0:], (0, EPB - E1), mode="wrap")])
    v2p = jnp.pad(v.reshape(N, 3 * DV), ((0, NP - N), (0, 0)))
    s_a, v_a = _scatter_half(m0, w0, rowp, jnp.zeros((NP, 96), jnp.float32),
                             0, E0 // CH)
    s_out_p, v_new96p = _scatter_half(m1, w1, rowp, v2p,
                                      E0 // CH, EPB // CH, prev=(s_a, v_a))
    s_new = _node_out(s, s_out_p[:N], W_s, b_s.reshape(1, DS),
                      ln_g.reshape(1, DS), ln_b.reshape(1, DS))
    return (s_new, v_new96p[:N].reshape(N, DV, 3))


# speedup vs baseline: 1.0431x; 1.0431x over previous
import functools

import jax
import jax.numpy as jnp
from jax import lax
from jax.experimental import pallas as pl
from jax.experimental.pallas import tpu as pltpu
from jax.experimental.pallas import tpu_sc as plsc

N = 10000
E = 320000
DS = 128
DV = 32

NC = 2
NS = 16
CH = 80

NP = 10240
EP = 4096 * CH
E0 = 204800
E1 = E - E0
EPB = EP - E0


def _dotT(x, w):
    return lax.dot_general(x, w, (((1,), (1,)), ((), ())),
                           preferred_element_type=jnp.float32)


def _dot(x, w):
    return lax.dot_general(x, w, (((1,), (0,)), ((), ())),
                           preferred_element_type=jnp.float32)


def _proj_body(s_ref, wa_ref, wb_ref, sa_ref, sb_ref):
    x = s_ref[...]
    sa_ref[...] = _dotT(x, wa_ref[...])
    sb_ref[...] = _dotT(x, wb_ref[...])


def _node_proj(s, W1a, W1b):
    B = 1000
    return pl.pallas_call(
        _proj_body,
        grid=(N // B,),
        in_specs=[pl.BlockSpec((B, DS), lambda i: (i, 0)),
                  pl.BlockSpec((DS, DS), lambda i: (0, 0)),
                  pl.BlockSpec((DS, DS), lambda i: (0, 0))],
        out_specs=[pl.BlockSpec((B, DS), lambda i: (i, 0)),
                   pl.BlockSpec((B, DS), lambda i: (i, 0))],
        out_shape=[jax.ShapeDtypeStruct((N, DS), jnp.float32),
                   jax.ShapeDtypeStruct((N, DS), jnp.float32)],
    )(s, W1a, W1b)


def _gather(sa, sb, row, col, e0, esz):
    mesh = plsc.VectorSubcoreMesh(core_axis_name="c", subcore_axis_name="s")
    per_w = esz // (NC * NS)
    n_ch = per_w // CH
    n_quad = n_ch // 4
    n_tail = n_ch % 4
    NR = 4

    @functools.partial(
        pl.kernel, mesh=mesh,
        out_type=jax.ShapeDtypeStruct((esz, DS), jnp.float32),
        scratch_types=[
            pltpu.VMEM((per_w,), jnp.int32),
            pltpu.VMEM((per_w,), jnp.int32),
            [pltpu.VMEM((CH, DS), jnp.float32) for _ in range(NR)],
            [pltpu.VMEM((CH, DS), jnp.float32) for _ in range(NR)],
            [pltpu.SemaphoreType.DMA for _ in range(NR)],
            [pltpu.SemaphoreType.DMA for _ in range(NR)],
        ],
    )
    def k(sa_hbm, sb_hbm, row_hbm, col_hbm, g_hbm,
          idxr, idxc, abufs, bbufs, insems, wrsems):
        cid = lax.axis_index("c")
        sid = lax.axis_index("s")
        wid = sid * NC + cid
        base = wid * per_w
        pltpu.sync_copy(row_hbm.at[pl.ds(e0 + base, per_w)], idxr)
        pltpu.sync_copy(col_hbm.at[pl.ds(e0 + base, per_w)], idxc)

        def issue(i, r):
            pltpu.async_copy(sa_hbm.at[idxr.at[pl.ds(i * CH, CH)]],
                             abufs[r], insems[r])
            pltpu.async_copy(sb_hbm.at[idxc.at[pl.ds(i * CH, CH)]],
                             bbufs[r], insems[r])

        def wait_in(r):
            pltpu.make_async_copy(sa_hbm.at[pl.ds(0, CH)], abufs[r],
                                  insems[r]).wait()
            pltpu.make_async_copy(sa_hbm.at[pl.ds(0, CH)], bbufs[r],
                                  insems[r]).wait()

        def wait_wr(r):
            pltpu.make_async_copy(abufs[r], g_hbm.at[pl.ds(0, CH)],
                                  wrsems[r]).wait()

        def add_write(i, r):
            abuf, bbuf = abufs[r], bbufs[r]

            def add_row(rr, c2):
                for c8 in range(DS // 16):
                    sl = pl.ds(c8 * 16, 16)
                    abuf[rr, sl] = abuf[rr, sl] + bbuf[rr, sl]
                return c2
            lax.fori_loop(0, CH, add_row, 0, unroll=8)
            pltpu.async_copy(abuf, g_hbm.at[pl.ds(base + i * CH, CH)],
                             wrsems[r])

        for r in range(NR - 1):
            issue(r, r)

        def body(t, carry):
            c = NR * t
            for kk in range(NR):
                r = kk
                rn = (kk + NR - 1) % NR
                wait_in(r)
                add_write(c + kk, r)
                ck = c + kk

                @pl.when((ck > 0) & (ck + NR - 1 < n_ch))
                def _():
                    wait_wr(rn)

                @pl.when(ck + NR - 1 < n_ch)
                def _():
                    issue(ck + NR - 1, rn)
            return carry
        lax.fori_loop(0, n_quad, body, 0)

        for ck in range(n_quad * NR, n_ch):
            wait_in(ck % NR)
            add_write(ck, ck % NR)
        for r in range(NR):
            wait_wr(r)

    return k(sa, sb, row, col)


def _edge_body_sz(nlive, g_ref, ea_ref, un_ref, w1c_ref, b1_ref, w2_ref,
                  b2_ref, wvb_ref, bvb_ref, p_ref, m_ref, w_ref):
    h = g_ref[...] + _dotT(ea_ref[...], w1c_ref[...]) + b1_ref[...]
    h = h * jax.nn.sigmoid(h)
    m = _dotT(h, w2_ref[...]) + b2_ref[...]
    sm = m * jax.nn.sigmoid(m)
    w = _dot(sm, wvb_ref[...]) + bvb_ref[...]
    u = _dot(un_ref[...], p_ref[...])
    B = m.shape[0]
    erow = pl.program_id(0) * B + lax.broadcasted_iota(jnp.int32, (B, 1), 0)
    live = erow < nlive
    m_ref[...] = jnp.where(live, m, 0.0)
    w_ref[...] = jnp.where(live, w * u, 0.0)


def _edge_mlp(g, ea, unit, W1c, b1, W2, b2, Wv_int, bv_int, P, blk0, esz, epad):
    B = 1024
    nlive_blk = (esz - 1) // B
    clampg = lambda i: (jnp.minimum(i, nlive_blk), 0)
    clampo = lambda i: (jnp.minimum(i, nlive_blk) + blk0, 0)
    const = lambda i: (0, 0)
    body = functools.partial(_edge_body_sz, esz)
    return pl.pallas_call(
        body,
        grid=(epad // B,),
        in_specs=[pl.BlockSpec((B, DS), clampg),
                  pl.BlockSpec((B, DS), clampo),
                  pl.BlockSpec((B, 3), clampo),
                  pl.BlockSpec((DS, DS), const),
                  pl.BlockSpec((1, DS), const),
                  pl.BlockSpec((DS, DS), const),
                  pl.BlockSpec((1, DS), const),
                  pl.BlockSpec((DS, DS), const),
                  pl.BlockSpec((1, DS), const),
                  pl.BlockSpec((3, DS), const)],
        out_specs=[pl.BlockSpec((B, DS), lambda i: (i, 0)),
                   pl.BlockSpec((B, DS), lambda i: (i, 0))],
        out_shape=[jax.ShapeDtypeStruct((epad, DS), jnp.float32),
                   jax.ShapeDtypeStruct((epad, DS), jnp.float32)],
    )(g, ea, unit, W1c, b1, W2, b2, Wv_int, bv_int, P)


def _scatter_half(m, w, rowp, v2p, c0, n_chunks, prev=None):
    mesh = plsc.VectorSubcoreMesh(core_axis_name="c", subcore_axis_name="s")
    per_t = n_chunks // NS
    n_tri = per_t // 3
    nb = NP // NS
    ZR = 32
    NR = 3
    final = prev is not None

    @functools.partial(
        pl.kernel, mesh=mesh,
        out_type=[jax.ShapeDtypeStruct((NP, DS), jnp.float32),
                  jax.ShapeDtypeStruct((NP, 96), jnp.float32)],
        scratch_types=[
            pltpu.VMEM_SHARED((NP, DS), jnp.float32),
            pltpu.VMEM((NR, CH), jnp.int32),
            [pltpu.VMEM((CH, DS), jnp.float32) for _ in range(NR)],
            pltpu.VMEM((ZR, DS), jnp.float32),
            pltpu.VMEM((ZR, DS), jnp.float32),
            pltpu.VMEM((ZR, 96), jnp.float32),
            pltpu.VMEM((ZR, 96), jnp.float32),
            [pltpu.SemaphoreType.DMA for _ in range(NR)],
            [pltpu.SemaphoreType.DMA for _ in range(NR)],
        ],
    )
    def k(m_hbm, w_hbm, row_hbm, v_hbm, *rest):
        if final:
            sprev_hbm, vprev_hbm = rest[0], rest[1]
            rest = rest[2:]
        (sout_hbm, vnew_hbm, acc, idxs, dats, stage, stage2,
         vstage, vstage2, insems, sssems) = rest
        cid = lax.axis_index("c")
        sid = lax.axis_index("s")
        zero16 = jnp.zeros((16,), jnp.float32)

        def zrow(r, c):
            for c8 in range(DS // 16):
                stage[r, pl.ds(c8 * 16, 16)] = zero16
            return c
        lax.fori_loop(0, ZR, zrow, 0)
        nbase = sid * nb

        def zblk(j, c):
            b0 = pl.multiple_of(nbase + j * ZR, 8)
            pltpu.async_copy(stage, acc.at[pl.ds(b0, ZR)], insems[0])
            return c
        lax.fori_loop(0, nb // ZR, zblk, 0)

        def zdrain(j, c):
            pltpu.make_async_copy(stage, acc.at[pl.ds(0, ZR)],
                                  insems[0]).wait()
            return c
        lax.fori_loop(0, nb // ZR, zdrain, 0)
        plsc.subcore_barrier()

        cbase = sid * per_t

        def issue(lc, r):
            gr = cbase + lc
            pltpu.async_copy(row_hbm.at[pl.ds((c0 + gr) * CH, CH)],
                             idxs.at[r], insems[r])

            @pl.when(cid == 0)
            def _():
                pltpu.async_copy(m_hbm.at[pl.ds(gr * CH, CH)], dats[r],
                                 insems[r])

            @pl.when(cid == 1)
            def _():
                pltpu.async_copy(w_hbm.at[pl.ds(gr * CH, CH)], dats[r],
                                 insems[r])

        def wait_in(r):
            pltpu.make_async_copy(row_hbm.at[pl.ds(0, CH)], idxs.at[r],
                                  insems[r]).wait()
            pltpu.make_async_copy(m_hbm.at[pl.ds(0, CH)], dats[r],
                                  insems[r]).wait()

        def scat(r):
            pltpu.async_copy(dats[r], acc.at[idxs.at[r]], sssems[r], add=True)

        def wait_ss(r):
            pltpu.make_async_copy(dats[r], acc.at[idxs.at[r]],
                                  sssems[r]).wait()

        for r in range(NR - 1):
            issue(r, r)

        def body(t, carry):
            c = NR * t
            for kk in range(NR):
                r = kk
                rn = (kk + NR - 1) % NR
                wait_in(r)
                scat(r)
                ck = c + kk

                @pl.when((ck > 0) & (ck + NR - 1 < per_t))
                def _():
                    wait_ss(rn)

                @pl.when(ck + NR - 1 < per_t)
                def _():
                    issue(ck + NR - 1, rn)
            return carry
        lax.fori_loop(0, n_tri, body, 0)

        for ck in range(n_tri * NR, per_t):
            wait_in(ck % NR)
            scat(ck % NR)
        for r in range(NR):
            wait_ss(r)
        plsc.subcore_barrier()

        @pl.when(cid == 0)
        def _():
            def sblk(j, c):
                b0 = pl.multiple_of(nbase + j * ZR, 8)
                pltpu.sync_copy(acc.at[pl.ds(b0, ZR)], stage)
                if final:
                    pltpu.sync_copy(sprev_hbm.at[pl.ds(b0, ZR)], stage2)

                    def srow(r, c2):
                        for vv in range(DS // 16):
                            sl = pl.ds(vv * 16, 16)
                            stage[r, sl] = stage[r, sl] + stage2[r, sl]
                        return c2
                    lax.fori_loop(0, ZR, srow, 0, unroll=4)
                pltpu.sync_copy(stage, sout_hbm.at[pl.ds(b0, ZR)])
                return c
            lax.fori_loop(0, nb // ZR, sblk, 0)

        @pl.when(cid == 1)
        def _():
            def vblk(j, c):
                b0 = pl.multiple_of(nbase + j * ZR, 8)
                pltpu.sync_copy(acc.at[pl.ds(b0, ZR)], stage)
                pltpu.sync_copy(v_hbm.at[pl.ds(b0, ZR)], vstage)
                if final:
                    pltpu.sync_copy(vprev_hbm.at[pl.ds(b0, ZR)], vstage2)

                def prow(r, c2):
                    for vv in range(6):
                        sl = pl.ds(vv * 16, 16)
                        x = vstage[r, sl] + stage[r, sl]
                        if final:
                            x = x + vstage2[r, sl]
                        vstage[r, sl] = x
                    return c2
                lax.fori_loop(0, ZR, prow, 0, unroll=4)
                pltpu.sync_copy(vstage, vnew_hbm.at[pl.ds(b0, ZR)])
                return c
            lax.fori_loop(0, nb // ZR, vblk, 0)

    if final:
        return k(m, w, rowp, v2p, prev[0], prev[1])
    return k(m, w, rowp, v2p)


def _node_body(s_ref, so_ref, ws_ref, bs_ref, g_ref, b_ref, out_ref):
    so = so_ref[...]
    sm = so * jax.nn.sigmoid(so)
    u = s_ref[...] + _dotT(sm, ws_ref[...]) + bs_ref[...]
    mu = jnp.mean(u, axis=1, keepdims=True)
    d = u - mu
    var = jnp.mean(d * d, axis=1, keepdims=True)
    out_ref[...] = d * lax.rsqrt(var + 1e-5) * g_ref[...] + b_ref[...]


def _node_out(s, s_out, W_s, bs, g, b):
    B = 1000
    return pl.pallas_call(
        _node_body,
        grid=(N // B,),
        in_specs=[pl.BlockSpec((B, DS), lambda i: (i, 0)),
                  pl.BlockSpec((B, DS), lambda i: (i, 0)),
                  pl.BlockSpec((DS, DS), lambda i: (0, 0)),
                  pl.BlockSpec((1, DS), lambda i: (0, 0)),
                  pl.BlockSpec((1, DS), lambda i: (0, 0)),
                  pl.BlockSpec((1, DS), lambda i: (0, 0))],
        out_specs=pl.BlockSpec((B, DS), lambda i: (i, 0)),
        out_shape=jax.ShapeDtypeStruct((N, DS), jnp.float32),
    )(s, s_out, W_s, bs, g, b)


def kernel(s, v, edge_index, edge_attr, edge_vec_unit,
           W_e1, b_e1, W_e2, b_e2, W_s, b_s, W_v, b_v, ln_g, ln_b):
    row = edge_index[0]
    col = edge_index[1]
    W1a = W_e1[:, :DS]
    W1b = W_e1[:, DS:2 * DS]
    W1c = W_e1[:, 2 * DS:]
    WvT = W_v.T
    jidx = jnp.arange(96) // 3
    Wv_int = jnp.concatenate(
        [WvT[:, jidx], jnp.zeros((DS, DV), jnp.float32)], axis=1)
    bv_int = jnp.pad(b_v[jidx], (0, DV)).reshape(1, DS)
    kcol = jnp.arange(96) % 3
    P = jnp.pad((kcol[None, :] == jnp.arange(3)[:, None]).astype(jnp.float32),
                ((0, 0), (0, DV)))

    sa, sb = _node_proj(s, W1a, W1b)
    mlp_w = (W1c, b_e1.reshape(1, DS), W_e2, b_e2.reshape(1, DS),
             Wv_int, bv_int, P)
    g0 = _gather(sa, sb, row, col, 0, E0)
    g1 = _gather(sa, sb, row, col, E0, E1)
    m0, w0 = _edge_mlp(g0, edge_attr, edge_vec_unit, *mlp_w, 0, E0, E0)
    m1, w1 = _edge_mlp(g1, edge_attr, edge_vec_unit, *mlp_w,
                       E0 // 1024, E1, EPB)
    rowp = jnp.concatenate(
        [row[:E0], jnp.pad(row[E0:], (0, EPB - E1), mode="wrap")])
    v2p = jnp.pad(v.reshape(N, 3 * DV), ((0, NP - N), (0, 0)))
    s_a, v_a = _scatter_half(m0, w0, rowp, jnp.zeros((NP, 96), jnp.float32),
                             0, E0 // CH)
    s_out_p, v_new96p = _scatter_half(m1, w1, rowp, v2p,
                                      E0 // CH, EPB // CH, prev=(s_a, v_a))
    s_new = _node_out(s, s_out_p[:N], W_s, b_s.reshape(1, DS),
                      ln_g.reshape(1, DS), ln_b.reshape(1, DS))
    return (s_new, v_new96p[:N].reshape(N, DV, 3))


# speedup vs baseline: 1.0843x; 1.0395x over previous
import functools

import jax
import jax.numpy as jnp
from jax import lax
from jax.experimental import pallas as pl
from jax.experimental.pallas import tpu as pltpu
from jax.experimental.pallas import tpu_sc as plsc

N = 10000
E = 320000
DS = 128
DV = 32

NC = 2
NS = 16
CH = 80

NP = 10240
EP = 4096 * CH
E0 = 163840
E1 = E - E0
EPB = EP - E0


def _dotT(x, w):
    return lax.dot_general(x, w, (((1,), (1,)), ((), ())),
                           preferred_element_type=jnp.float32)


def _dot(x, w):
    return lax.dot_general(x, w, (((1,), (0,)), ((), ())),
                           preferred_element_type=jnp.float32)


def _proj_body(s_ref, wa_ref, wb_ref, sa_ref, sb_ref):
    x = s_ref[...]
    sa_ref[...] = _dotT(x, wa_ref[...])
    sb_ref[...] = _dotT(x, wb_ref[...])


def _node_proj(s, W1a, W1b):
    B = 1000
    return pl.pallas_call(
        _proj_body,
        grid=(N // B,),
        in_specs=[pl.BlockSpec((B, DS), lambda i: (i, 0)),
                  pl.BlockSpec((DS, DS), lambda i: (0, 0)),
                  pl.BlockSpec((DS, DS), lambda i: (0, 0))],
        out_specs=[pl.BlockSpec((B, DS), lambda i: (i, 0)),
                   pl.BlockSpec((B, DS), lambda i: (i, 0))],
        out_shape=[jax.ShapeDtypeStruct((N, DS), jnp.float32),
                   jax.ShapeDtypeStruct((N, DS), jnp.float32)],
    )(s, W1a, W1b)


def _gather(sa, sb, row, col, e0, esz):
    mesh = plsc.VectorSubcoreMesh(core_axis_name="c", subcore_axis_name="s")
    per_w = esz // (NC * NS)
    n_ch = per_w // CH
    n_quad = n_ch // 4
    n_tail = n_ch % 4
    NR = 4

    @functools.partial(
        pl.kernel, mesh=mesh,
        out_type=jax.ShapeDtypeStruct((esz, DS), jnp.float32),
        scratch_types=[
            pltpu.VMEM((per_w,), jnp.int32),
            pltpu.VMEM((per_w,), jnp.int32),
            [pltpu.VMEM((CH, DS), jnp.float32) for _ in range(NR)],
            [pltpu.VMEM((CH, DS), jnp.float32) for _ in range(NR)],
            [pltpu.SemaphoreType.DMA for _ in range(NR)],
            [pltpu.SemaphoreType.DMA for _ in range(NR)],
        ],
    )
    def k(sa_hbm, sb_hbm, row_hbm, col_hbm, g_hbm,
          idxr, idxc, abufs, bbufs, insems, wrsems):
        cid = lax.axis_index("c")
        sid = lax.axis_index("s")
        wid = sid * NC + cid
        base = wid * per_w
        pltpu.sync_copy(row_hbm.at[pl.ds(e0 + base, per_w)], idxr)
        pltpu.sync_copy(col_hbm.at[pl.ds(e0 + base, per_w)], idxc)

        def issue(i, r):
            pltpu.async_copy(sa_hbm.at[idxr.at[pl.ds(i * CH, CH)]],
                             abufs[r], insems[r])
            pltpu.async_copy(sb_hbm.at[idxc.at[pl.ds(i * CH, CH)]],
                             bbufs[r], insems[r])

        def wait_in(r):
            pltpu.make_async_copy(sa_hbm.at[pl.ds(0, CH)], abufs[r],
                                  insems[r]).wait()
            pltpu.make_async_copy(sa_hbm.at[pl.ds(0, CH)], bbufs[r],
                                  insems[r]).wait()

        def wait_wr(r):
            pltpu.make_async_copy(abufs[r], g_hbm.at[pl.ds(0, CH)],
                                  wrsems[r]).wait()

        def add_write(i, r):
            abuf, bbuf = abufs[r], bbufs[r]

            def add_row(rr, c2):
                for c8 in range(DS // 16):
                    sl = pl.ds(c8 * 16, 16)
                    abuf[rr, sl] = abuf[rr, sl] + bbuf[rr, sl]
                return c2
            lax.fori_loop(0, CH, add_row, 0, unroll=8)
            pltpu.async_copy(abuf, g_hbm.at[pl.ds(base + i * CH, CH)],
                             wrsems[r])

        for r in range(NR - 1):
            issue(r, r)

        def body(t, carry):
            c = NR * t
            for kk in range(NR):
                r = kk
                rn = (kk + NR - 1) % NR
                wait_in(r)
                add_write(c + kk, r)
                ck = c + kk

                @pl.when((ck > 0) & (ck + NR - 1 < n_ch))
                def _():
                    wait_wr(rn)

                @pl.when(ck + NR - 1 < n_ch)
                def _():
                    issue(ck + NR - 1, rn)
            return carry
        lax.fori_loop(0, n_quad, body, 0)

        for ck in range(n_quad * NR, n_ch):
            wait_in(ck % NR)
            add_write(ck, ck % NR)
        for r in range(NR):
            wait_wr(r)

    return k(sa, sb, row, col)


def _edge_body_sz(nlive, g_ref, ea_ref, un_ref, w1c_ref, b1_ref, w2_ref,
                  b2_ref, wvb_ref, bvb_ref, p_ref, m_ref, w_ref):
    h = g_ref[...] + _dotT(ea_ref[...], w1c_ref[...]) + b1_ref[...]
    h = h * jax.nn.sigmoid(h)
    m = _dotT(h, w2_ref[...]) + b2_ref[...]
    sm = m * jax.nn.sigmoid(m)
    w = _dot(sm, wvb_ref[...]) + bvb_ref[...]
    u = _dot(un_ref[...], p_ref[...])
    B = m.shape[0]
    erow = pl.program_id(0) * B + lax.broadcasted_iota(jnp.int32, (B, 1), 0)
    live = erow < nlive
    m_ref[...] = jnp.where(live, m, 0.0)
    w_ref[...] = jnp.where(live, w * u, 0.0)


def _edge_mlp(g, ea, unit, W1c, b1, W2, b2, Wv_int, bv_int, P, blk0, esz, epad):
    B = 1024
    nlive_blk = (esz - 1) // B
    clampg = lambda i: (jnp.minimum(i, nlive_blk), 0)
    clampo = lambda i: (jnp.minimum(i, nlive_blk) + blk0, 0)
    const = lambda i: (0, 0)
    body = functools.partial(_edge_body_sz, esz)
    return pl.pallas_call(
        body,
        grid=(epad // B,),
        in_specs=[pl.BlockSpec((B, DS), clampg),
                  pl.BlockSpec((B, DS), clampo),
                  pl.BlockSpec((B, 3), clampo),
                  pl.BlockSpec((DS, DS), const),
                  pl.BlockSpec((1, DS), const),
                  pl.BlockSpec((DS, DS), const),
                  pl.BlockSpec((1, DS), const),
                  pl.BlockSpec((DS, DS), const),
                  pl.BlockSpec((1, DS), const),
                  pl.BlockSpec((3, DS), const)],
        out_specs=[pl.BlockSpec((B, DS), lambda i: (i, 0)),
                   pl.BlockSpec((B, DS), lambda i: (i, 0))],
        out_shape=[jax.ShapeDtypeStruct((epad, DS), jnp.float32),
                   jax.ShapeDtypeStruct((epad, DS), jnp.float32)],
    )(g, ea, unit, W1c, b1, W2, b2, Wv_int, bv_int, P)


def _scatter_half(m, w, rowp, v2p, c0, n_chunks, prev=None):
    mesh = plsc.VectorSubcoreMesh(core_axis_name="c", subcore_axis_name="s")
    per_t = n_chunks // NS
    n_tri = per_t // 3
    nb = NP // NS
    ZR = 32
    NR = 3
    final = prev is not None

    @functools.partial(
        pl.kernel, mesh=mesh,
        out_type=[jax.ShapeDtypeStruct((NP, DS), jnp.float32),
                  jax.ShapeDtypeStruct((NP, 96), jnp.float32)],
        scratch_types=[
            pltpu.VMEM_SHARED((NP, DS), jnp.float32),
            pltpu.VMEM((NR, CH), jnp.int32),
            [pltpu.VMEM((CH, DS), jnp.float32) for _ in range(NR)],
            pltpu.VMEM((ZR, DS), jnp.float32),
            pltpu.VMEM((ZR, DS), jnp.float32),
            pltpu.VMEM((ZR, 96), jnp.float32),
            pltpu.VMEM((ZR, 96), jnp.float32),
            [pltpu.SemaphoreType.DMA for _ in range(NR)],
            [pltpu.SemaphoreType.DMA for _ in range(NR)],
        ],
    )
    def k(m_hbm, w_hbm, row_hbm, v_hbm, *rest):
        if final:
            sprev_hbm, vprev_hbm = rest[0], rest[1]
            rest = rest[2:]
        (sout_hbm, vnew_hbm, acc, idxs, dats, stage, stage2,
         vstage, vstage2, insems, sssems) = rest
        cid = lax.axis_index("c")
        sid = lax.axis_index("s")
        zero16 = jnp.zeros((16,), jnp.float32)

        def zrow(r, c):
            for c8 in range(DS // 16):
                stage[r, pl.ds(c8 * 16, 16)] = zero16
            return c
        lax.fori_loop(0, ZR, zrow, 0)
        nbase = sid * nb

        def zblk(j, c):
            b0 = pl.multiple_of(nbase + j * ZR, 8)
            pltpu.async_copy(stage, acc.at[pl.ds(b0, ZR)], insems[0])
            return c
        lax.fori_loop(0, nb // ZR, zblk, 0)

        def zdrain(j, c):
            pltpu.make_async_copy(stage, acc.at[pl.ds(0, ZR)],
                                  insems[0]).wait()
            return c
        lax.fori_loop(0, nb // ZR, zdrain, 0)
        plsc.subcore_barrier()

        cbase = sid * per_t

        def issue(lc, r):
            gr = cbase + lc
            pltpu.async_copy(row_hbm.at[pl.ds((c0 + gr) * CH, CH)],
                             idxs.at[r], insems[r])

            @pl.when(cid == 0)
            def _():
                pltpu.async_copy(m_hbm.at[pl.ds(gr * CH, CH)], dats[r],
                                 insems[r])

            @pl.when(cid == 1)
            def _():
                pltpu.async_copy(w_hbm.at[pl.ds(gr * CH, CH)], dats[r],
                                 insems[r])

        def wait_in(r):
            pltpu.make_async_copy(row_hbm.at[pl.ds(0, CH)], idxs.at[r],
                                  insems[r]).wait()
            pltpu.make_async_copy(m_hbm.at[pl.ds(0, CH)], dats[r],
                                  insems[r]).wait()

        def scat(r):
            pltpu.async_copy(dats[r], acc.at[idxs.at[r]], sssems[r], add=True)

        def wait_ss(r):
            pltpu.make_async_copy(dats[r], acc.at[idxs.at[r]],
                                  sssems[r]).wait()

        for r in range(NR - 1):
            issue(r, r)

        def body(t, carry):
            c = NR * t
            for kk in range(NR):
                r = kk
                rn = (kk + NR - 1) % NR
                wait_in(r)
                scat(r)
                ck = c + kk

                @pl.when((ck > 0) & (ck + NR - 1 < per_t))
                def _():
                    wait_ss(rn)

                @pl.when(ck + NR - 1 < per_t)
                def _():
                    issue(ck + NR - 1, rn)
            return carry
        lax.fori_loop(0, n_tri, body, 0)

        for ck in range(n_tri * NR, per_t):
            wait_in(ck % NR)
            scat(ck % NR)
        for r in range(NR):
            wait_ss(r)
        plsc.subcore_barrier()

        @pl.when(cid == 0)
        def _():
            def sblk(j, c):
                b0 = pl.multiple_of(nbase + j * ZR, 8)
                pltpu.sync_copy(acc.at[pl.ds(b0, ZR)], stage)
                if final:
                    pltpu.sync_copy(sprev_hbm.at[pl.ds(b0, ZR)], stage2)

                    def srow(r, c2):
                        for vv in range(DS // 16):
                            sl = pl.ds(vv * 16, 16)
                            stage[r, sl] = stage[r, sl] + stage2[r, sl]
                        return c2
                    lax.fori_loop(0, ZR, srow, 0, unroll=4)
                pltpu.sync_copy(stage, sout_hbm.at[pl.ds(b0, ZR)])
                return c
            lax.fori_loop(0, nb // ZR, sblk, 0)

        @pl.when(cid == 1)
        def _():
            def vblk(j, c):
                b0 = pl.multiple_of(nbase + j * ZR, 8)
                pltpu.sync_copy(acc.at[pl.ds(b0, ZR)], stage)
                pltpu.sync_copy(v_hbm.at[pl.ds(b0, ZR)], vstage)
                if final:
                    pltpu.sync_copy(vprev_hbm.at[pl.ds(b0, ZR)], vstage2)

                def prow(r, c2):
                    for vv in range(6):
                        sl = pl.ds(vv * 16, 16)
                        x = vstage[r, sl] + stage[r, sl]
                        if final:
                            x = x + vstage2[r, sl]
                        vstage[r, sl] = x
                    return c2
                lax.fori_loop(0, ZR, prow, 0, unroll=4)
                pltpu.sync_copy(vstage, vnew_hbm.at[pl.ds(b0, ZR)])
                return c
            lax.fori_loop(0, nb // ZR, vblk, 0)

    if final:
        return k(m, w, rowp, v2p, prev[0], prev[1])
    return k(m, w, rowp, v2p)


def _node_body(s_ref, so_ref, ws_ref, bs_ref, g_ref, b_ref, out_ref):
    so = so_ref[...]
    sm = so * jax.nn.sigmoid(so)
    u = s_ref[...] + _dotT(sm, ws_ref[...]) + bs_ref[...]
    mu = jnp.mean(u, axis=1, keepdims=True)
    d = u - mu
    var = jnp.mean(d * d, axis=1, keepdims=True)
    out_ref[...] = d * lax.rsqrt(var + 1e-5) * g_ref[...] + b_ref[...]


def _node_out(s, s_out, W_s, bs, g, b):
    B = 1000
    return pl.pallas_call(
        _node_body,
        grid=(N // B,),
        in_specs=[pl.BlockSpec((B, DS), lambda i: (i, 0)),
                  pl.BlockSpec((B, DS), lambda i: (i, 0)),
                  pl.BlockSpec((DS, DS), lambda i: (0, 0)),
                  pl.BlockSpec((1, DS), lambda i: (0, 0)),
                  pl.BlockSpec((1, DS), lambda i: (0, 0)),
                  pl.BlockSpec((1, DS), lambda i: (0, 0))],
        out_specs=pl.BlockSpec((B, DS), lambda i: (i, 0)),
        out_shape=jax.ShapeDtypeStruct((N, DS), jnp.float32),
    )(s, s_out, W_s, bs, g, b)


def kernel(s, v, edge_index, edge_attr, edge_vec_unit,
           W_e1, b_e1, W_e2, b_e2, W_s, b_s, W_v, b_v, ln_g, ln_b):
    row = edge_index[0]
    col = edge_index[1]
    W1a = W_e1[:, :DS]
    W1b = W_e1[:, DS:2 * DS]
    W1c = W_e1[:, 2 * DS:]
    WvT = W_v.T
    jidx = jnp.arange(96) // 3
    Wv_int = jnp.concatenate(
        [WvT[:, jidx], jnp.zeros((DS, DV), jnp.float32)], axis=1)
    bv_int = jnp.pad(b_v[jidx], (0, DV)).reshape(1, DS)
    kcol = jnp.arange(96) % 3
    P = jnp.pad((kcol[None, :] == jnp.arange(3)[:, None]).astype(jnp.float32),
                ((0, 0), (0, DV)))

    sa, sb = _node_proj(s, W1a, W1b)
    mlp_w = (W1c, b_e1.reshape(1, DS), W_e2, b_e2.reshape(1, DS),
             Wv_int, bv_int, P)
    g0 = _gather(sa, sb, row, col, 0, E0)
    g1 = _gather(sa, sb, row, col, E0, E1)
    m0, w0 = _edge_mlp(g0, edge_attr, edge_vec_unit, *mlp_w, 0, E0, E0)
    m1, w1 = _edge_mlp(g1, edge_attr, edge_vec_unit, *mlp_w,
                       E0 // 1024, E1, EPB)
    rowp = jnp.concatenate(
        [row[:E0], jnp.pad(row[E0:], (0, EPB - E1), mode="wrap")])
    v2p = jnp.pad(v.reshape(N, 3 * DV), ((0, NP - N), (0, 0)))
    s_a, v_a = _scatter_half(m0, w0, rowp, jnp.zeros((NP, 96), jnp.float32),
                             0, E0 // CH)
    s_out_p, v_new96p = _scatter_half(m1, w1, rowp, v2p,
                                      E0 // CH, EPB // CH, prev=(s_a, v_a))
    s_new = _node_out(s, s_out_p[:N], W_s, b_s.reshape(1, DS),
                      ln_g.reshape(1, DS), ln_b.reshape(1, DS))
    return (s_new, v_new96p[:N].reshape(N, DV, 3))


# speedup vs baseline: 1.0951x; 1.0100x over previous
import functools

import jax
import jax.numpy as jnp
from jax import lax
from jax.experimental import pallas as pl
from jax.experimental.pallas import tpu as pltpu
from jax.experimental.pallas import tpu_sc as plsc

N = 10000
E = 320000
DS = 128
DV = 32

NC = 2
NS = 16
CH = 80

NP = 10240
EP = 4096 * CH
E0 = 143360
E1 = E - E0
EPB = EP - E0


def _dotT(x, w):
    return lax.dot_general(x, w, (((1,), (1,)), ((), ())),
                           preferred_element_type=jnp.float32)


def _dot(x, w):
    return lax.dot_general(x, w, (((1,), (0,)), ((), ())),
                           preferred_element_type=jnp.float32)


def _proj_body(s_ref, wa_ref, wb_ref, sa_ref, sb_ref):
    x = s_ref[...]
    sa_ref[...] = _dotT(x, wa_ref[...])
    sb_ref[...] = _dotT(x, wb_ref[...])


def _node_proj(s, W1a, W1b):
    B = 1000
    return pl.pallas_call(
        _proj_body,
        grid=(N // B,),
        in_specs=[pl.BlockSpec((B, DS), lambda i: (i, 0)),
                  pl.BlockSpec((DS, DS), lambda i: (0, 0)),
                  pl.BlockSpec((DS, DS), lambda i: (0, 0))],
        out_specs=[pl.BlockSpec((B, DS), lambda i: (i, 0)),
                   pl.BlockSpec((B, DS), lambda i: (i, 0))],
        out_shape=[jax.ShapeDtypeStruct((N, DS), jnp.float32),
                   jax.ShapeDtypeStruct((N, DS), jnp.float32)],
    )(s, W1a, W1b)


def _gather(sa, sb, row, col, e0, esz):
    mesh = plsc.VectorSubcoreMesh(core_axis_name="c", subcore_axis_name="s")
    per_w = esz // (NC * NS)
    n_ch = per_w // CH
    n_quad = n_ch // 4
    n_tail = n_ch % 4
    NR = 4

    @functools.partial(
        pl.kernel, mesh=mesh,
        out_type=jax.ShapeDtypeStruct((esz, DS), jnp.float32),
        scratch_types=[
            pltpu.VMEM((per_w,), jnp.int32),
            pltpu.VMEM((per_w,), jnp.int32),
            [pltpu.VMEM((CH, DS), jnp.float32) for _ in range(NR)],
            [pltpu.VMEM((CH, DS), jnp.float32) for _ in range(NR)],
            [pltpu.SemaphoreType.DMA for _ in range(NR)],
            [pltpu.SemaphoreType.DMA for _ in range(NR)],
        ],
    )
    def k(sa_hbm, sb_hbm, row_hbm, col_hbm, g_hbm,
          idxr, idxc, abufs, bbufs, insems, wrsems):
        cid = lax.axis_index("c")
        sid = lax.axis_index("s")
        wid = sid * NC + cid
        base = wid * per_w
        pltpu.sync_copy(row_hbm.at[pl.ds(e0 + base, per_w)], idxr)
        pltpu.sync_copy(col_hbm.at[pl.ds(e0 + base, per_w)], idxc)

        def issue(i, r):
            pltpu.async_copy(sa_hbm.at[idxr.at[pl.ds(i * CH, CH)]],
                             abufs[r], insems[r])
            pltpu.async_copy(sb_hbm.at[idxc.at[pl.ds(i * CH, CH)]],
                             bbufs[r], insems[r])

        def wait_in(r):
            pltpu.make_async_copy(sa_hbm.at[pl.ds(0, CH)], abufs[r],
                                  insems[r]).wait()
            pltpu.make_async_copy(sa_hbm.at[pl.ds(0, CH)], bbufs[r],
                                  insems[r]).wait()

        def wait_wr(r):
            pltpu.make_async_copy(abufs[r], g_hbm.at[pl.ds(0, CH)],
                                  wrsems[r]).wait()

        def add_write(i, r):
            abuf, bbuf = abufs[r], bbufs[r]

            def add_row(rr, c2):
                for c8 in range(DS // 16):
                    sl = pl.ds(c8 * 16, 16)
                    abuf[rr, sl] = abuf[rr, sl] + bbuf[rr, sl]
                return c2
            lax.fori_loop(0, CH, add_row, 0, unroll=8)
            pltpu.async_copy(abuf, g_hbm.at[pl.ds(base + i * CH, CH)],
                             wrsems[r])

        for r in range(NR - 1):
            issue(r, r)

        def body(t, carry):
            c = NR * t
            for kk in range(NR):
                r = kk
                rn = (kk + NR - 1) % NR
                wait_in(r)
                add_write(c + kk, r)
                ck = c + kk

                @pl.when((ck > 0) & (ck + NR - 1 < n_ch))
                def _():
                    wait_wr(rn)

                @pl.when(ck + NR - 1 < n_ch)
                def _():
                    issue(ck + NR - 1, rn)
            return carry
        lax.fori_loop(0, n_quad, body, 0)

        for ck in range(n_quad * NR, n_ch):
            wait_in(ck % NR)
            add_write(ck, ck % NR)
        for r in range(NR):
            wait_wr(r)

    return k(sa, sb, row, col)


def _edge_body_sz(nlive, g_ref, ea_ref, un_ref, w1c_ref, b1_ref, w2_ref,
                  b2_ref, wvb_ref, bvb_ref, p_ref, m_ref, w_ref):
    h = g_ref[...] + _dotT(ea_ref[...], w1c_ref[...]) + b1_ref[...]
    h = h * jax.nn.sigmoid(h)
    m = _dotT(h, w2_ref[...]) + b2_ref[...]
    sm = m * jax.nn.sigmoid(m)
    w = _dot(sm, wvb_ref[...]) + bvb_ref[...]
    u = _dot(un_ref[...], p_ref[...])
    B = m.shape[0]
    erow = pl.program_id(0) * B + lax.broadcasted_iota(jnp.int32, (B, 1), 0)
    live = erow < nlive
    m_ref[...] = jnp.where(live, m, 0.0)
    w_ref[...] = jnp.where(live, w * u, 0.0)


def _edge_mlp(g, ea, unit, W1c, b1, W2, b2, Wv_int, bv_int, P, blk0, esz, epad):
    B = 1024
    nlive_blk = (esz - 1) // B
    clampg = lambda i: (jnp.minimum(i, nlive_blk), 0)
    clampo = lambda i: (jnp.minimum(i, nlive_blk) + blk0, 0)
    const = lambda i: (0, 0)
    body = functools.partial(_edge_body_sz, esz)
    return pl.pallas_call(
        body,
        grid=(epad // B,),
        in_specs=[pl.BlockSpec((B, DS), clampg),
                  pl.BlockSpec((B, DS), clampo),
                  pl.BlockSpec((B, 3), clampo),
                  pl.BlockSpec((DS, DS), const),
                  pl.BlockSpec((1, DS), const),
                  pl.BlockSpec((DS, DS), const),
                  pl.BlockSpec((1, DS), const),
                  pl.BlockSpec((DS, DS), const),
                  pl.BlockSpec((1, DS), const),
                  pl.BlockSpec((3, DS), const)],
        out_specs=[pl.BlockSpec((B, DS), lambda i: (i, 0)),
                   pl.BlockSpec((B, DS), lambda i: (i, 0))],
        out_shape=[jax.ShapeDtypeStruct((epad, DS), jnp.float32),
                   jax.ShapeDtypeStruct((epad, DS), jnp.float32)],
    )(g, ea, unit, W1c, b1, W2, b2, Wv_int, bv_int, P)


def _scatter_half(m, w, rowp, v2p, c0, n_chunks, prev=None):
    mesh = plsc.VectorSubcoreMesh(core_axis_name="c", subcore_axis_name="s")
    per_t = n_chunks // NS
    n_tri = per_t // 3
    nb = NP // NS
    ZR = 32
    NR = 3
    final = prev is not None

    @functools.partial(
        pl.kernel, mesh=mesh,
        out_type=[jax.ShapeDtypeStruct((NP, DS), jnp.float32),
                  jax.ShapeDtypeStruct((NP, 96), jnp.float32)],
        scratch_types=[
            pltpu.VMEM_SHARED((NP, DS), jnp.float32),
            pltpu.VMEM((NR, CH), jnp.int32),
            [pltpu.VMEM((CH, DS), jnp.float32) for _ in range(NR)],
            pltpu.VMEM((ZR, DS), jnp.float32),
            pltpu.VMEM((ZR, DS), jnp.float32),
            pltpu.VMEM((ZR, 96), jnp.float32),
            pltpu.VMEM((ZR, 96), jnp.float32),
            [pltpu.SemaphoreType.DMA for _ in range(NR)],
            [pltpu.SemaphoreType.DMA for _ in range(NR)],
        ],
    )
    def k(m_hbm, w_hbm, row_hbm, v_hbm, *rest):
        if final:
            sprev_hbm, vprev_hbm = rest[0], rest[1]
            rest = rest[2:]
        (sout_hbm, vnew_hbm, acc, idxs, dats, stage, stage2,
         vstage, vstage2, insems, sssems) = rest
        cid = lax.axis_index("c")
        sid = lax.axis_index("s")
        zero16 = jnp.zeros((16,), jnp.float32)

        def zrow(r, c):
            for c8 in range(DS // 16):
                stage[r, pl.ds(c8 * 16, 16)] = zero16
            return c
        lax.fori_loop(0, ZR, zrow, 0)
        nbase = sid * nb

        def zblk(j, c):
            b0 = pl.multiple_of(nbase + j * ZR, 8)
            pltpu.async_copy(stage, acc.at[pl.ds(b0, ZR)], insems[0])
            return c
        lax.fori_loop(0, nb // ZR, zblk, 0)

        def zdrain(j, c):
            pltpu.make_async_copy(stage, acc.at[pl.ds(0, ZR)],
                                  insems[0]).wait()
            return c
        lax.fori_loop(0, nb // ZR, zdrain, 0)
        plsc.subcore_barrier()

        cbase = sid * per_t

        def issue(lc, r):
            gr = cbase + lc
            pltpu.async_copy(row_hbm.at[pl.ds((c0 + gr) * CH, CH)],
                             idxs.at[r], insems[r])

            @pl.when(cid == 0)
            def _():
                pltpu.async_copy(m_hbm.at[pl.ds(gr * CH, CH)], dats[r],
                                 insems[r])

            @pl.when(cid == 1)
            def _():
                pltpu.async_copy(w_hbm.at[pl.ds(gr * CH, CH)], dats[r],
                                 insems[r])

        def wait_in(r):
            pltpu.make_async_copy(row_hbm.at[pl.ds(0, CH)], idxs.at[r],
                                  insems[r]).wait()
            pltpu.make_async_copy(m_hbm.at[pl.ds(0, CH)], dats[r],
                                  insems[r]).wait()

        def scat(r):
            pltpu.async_copy(dats[r], acc.at[idxs.at[r]], sssems[r], add=True)

        def wait_ss(r):
            pltpu.make_async_copy(dats[r], acc.at[idxs.at[r]],
                                  sssems[r]).wait()

        for r in range(NR - 1):
            issue(r, r)

        def body(t, carry):
            c = NR * t
            for kk in range(NR):
                r = kk
                rn = (kk + NR - 1) % NR
                wait_in(r)
                scat(r)
                ck = c + kk

                @pl.when((ck > 0) & (ck + NR - 1 < per_t))
                def _():
                    wait_ss(rn)

                @pl.when(ck + NR - 1 < per_t)
                def _():
                    issue(ck + NR - 1, rn)
            return carry
        lax.fori_loop(0, n_tri, body, 0)

        for ck in range(n_tri * NR, per_t):
            wait_in(ck % NR)
            scat(ck % NR)
        for r in range(NR):
            wait_ss(r)
        plsc.subcore_barrier()

        @pl.when(cid == 0)
        def _():
            def sblk(j, c):
                b0 = pl.multiple_of(nbase + j * ZR, 8)
                pltpu.sync_copy(acc.at[pl.ds(b0, ZR)], stage)
                if final:
                    pltpu.sync_copy(sprev_hbm.at[pl.ds(b0, ZR)], stage2)

                    def srow(r, c2):
                        for vv in range(DS // 16):
                            sl = pl.ds(vv * 16, 16)
                            stage[r, sl] = stage[r, sl] + stage2[r, sl]
                        return c2
                    lax.fori_loop(0, ZR, srow, 0, unroll=4)
                pltpu.sync_copy(stage, sout_hbm.at[pl.ds(b0, ZR)])
                return c
            lax.fori_loop(0, nb // ZR, sblk, 0)

        @pl.when(cid == 1)
        def _():
            def vblk(j, c):
                b0 = pl.multiple_of(nbase + j * ZR, 8)
                pltpu.sync_copy(acc.at[pl.ds(b0, ZR)], stage)
                pltpu.sync_copy(v_hbm.at[pl.ds(b0, ZR)], vstage)
                if final:
                    pltpu.sync_copy(vprev_hbm.at[pl.ds(b0, ZR)], vstage2)

                def prow(r, c2):
                    for vv in range(6):
                        sl = pl.ds(vv * 16, 16)
                        x = vstage[r, sl] + stage[r, sl]
                        if final:
                            x = x + vstage2[r, sl]
                        vstage[r, sl] = x
                    return c2
                lax.fori_loop(0, ZR, prow, 0, unroll=4)
                pltpu.sync_copy(vstage, vnew_hbm.at[pl.ds(b0, ZR)])
                return c
            lax.fori_loop(0, nb // ZR, vblk, 0)

    if final:
        return k(m, w, rowp, v2p, prev[0], prev[1])
    return k(m, w, rowp, v2p)


def _node_body(s_ref, so_ref, ws_ref, bs_ref, g_ref, b_ref, out_ref):
    so = so_ref[...]
    sm = so * jax.nn.sigmoid(so)
    u = s_ref[...] + _dotT(sm, ws_ref[...]) + bs_ref[...]
    mu = jnp.mean(u, axis=1, keepdims=True)
    d = u - mu
    var = jnp.mean(d * d, axis=1, keepdims=True)
    out_ref[...] = d * lax.rsqrt(var + 1e-5) * g_ref[...] + b_ref[...]


def _node_out(s, s_out, W_s, bs, g, b):
    B = 1000
    return pl.pallas_call(
        _node_body,
        grid=(N // B,),
        in_specs=[pl.BlockSpec((B, DS), lambda i: (i, 0)),
                  pl.BlockSpec((B, DS), lambda i: (i, 0)),
                  pl.BlockSpec((DS, DS), lambda i: (0, 0)),
                  pl.BlockSpec((1, DS), lambda i: (0, 0)),
                  pl.BlockSpec((1, DS), lambda i: (0, 0)),
                  pl.BlockSpec((1, DS), lambda i: (0, 0))],
        out_specs=pl.BlockSpec((B, DS), lambda i: (i, 0)),
        out_shape=jax.ShapeDtypeStruct((N, DS), jnp.float32),
    )(s, s_out, W_s, bs, g, b)


def kernel(s, v, edge_index, edge_attr, edge_vec_unit,
           W_e1, b_e1, W_e2, b_e2, W_s, b_s, W_v, b_v, ln_g, ln_b):
    row = edge_index[0]
    col = edge_index[1]
    W1a = W_e1[:, :DS]
    W1b = W_e1[:, DS:2 * DS]
    W1c = W_e1[:, 2 * DS:]
    WvT = W_v.T
    jidx = jnp.arange(96) // 3
    Wv_int = jnp.concatenate(
        [WvT[:, jidx], jnp.zeros((DS, DV), jnp.float32)], axis=1)
    bv_int = jnp.pad(b_v[jidx], (0, DV)).reshape(1, DS)
    kcol = jnp.arange(96) % 3
    P = jnp.pad((kcol[None, :] == jnp.arange(3)[:, None]).astype(jnp.float32),
                ((0, 0), (0, DV)))

    sa, sb = _node_proj(s, W1a, W1b)
    mlp_w = (W1c, b_e1.reshape(1, DS), W_e2, b_e2.reshape(1, DS),
             Wv_int, bv_int, P)
    g0 = _gather(sa, sb, row, col, 0, E0)
    g1 = _gather(sa, sb, row, col, E0, E1)
    m0, w0 = _edge_mlp(g0, edge_attr, edge_vec_unit, *mlp_w, 0, E0, E0)
    m1, w1 = _edge_mlp(g1, edge_attr, edge_vec_unit, *mlp_w,
                       E0 // 1024, E1, EPB)
    rowp = jnp.concatenate(
        [row[:E0], jnp.pad(row[E0:], (0, EPB - E1), mode="wrap")])
    v2p = jnp.pad(v.reshape(N, 3 * DV), ((0, NP - N), (0, 0)))
    s_a, v_a = _scatter_half(m0, w0, rowp, jnp.zeros((NP, 96), jnp.float32),
                             0, E0 // CH)
    s_out_p, v_new96p = _scatter_half(m1, w1, rowp, v2p,
                                      E0 // CH, EPB // CH, prev=(s_a, v_a))
    s_new = _node_out(s, s_out_p[:N], W_s, b_s.reshape(1, DS),
                      ln_g.reshape(1, DS), ln_b.reshape(1, DS))
    return (s_new, v_new96p[:N].reshape(N, DV, 3))
